# baseline (speedup 1.0000x reference)
import functools

import jax
import jax.numpy as jnp
from jax import lax
from jax.experimental import pallas as pl
from jax.experimental.pallas import tpu as pltpu
from jax.experimental.pallas import tpu_sc as plsc

D = 768
B = 4
T = 2048
N = B * T
EPS = 1e-6
NC, NS, L = 2, 16, 16
NW = NC * NS
TW = T // NW
C = 32
NCHUNK = (B * TW) // C
NJ = D // L


def _rsqrt_scalar(a):
    i = lax.bitcast_convert_type(a, jnp.int32)
    i = jnp.int32(0x5F3759DF) - (i >> 1)
    y = lax.bitcast_convert_type(i, jnp.float32)
    half_a = 0.5 * a
    for _ in range(2):
        y = y * (1.5 - half_a * y * y)
    return y


_mesh = plsc.VectorSubcoreMesh(core_axis_name="c", subcore_axis_name="s")


@functools.partial(
    pl.kernel,
    mesh=_mesh,
    out_type=jax.ShapeDtypeStruct((N, D), jnp.float32),
    scratch_types=[
        pltpu.VMEM((NCHUNK, C), jnp.int32),
        pltpu.VMEM((C, D), jnp.float32),
        pltpu.VMEM((C, D), jnp.float32),
        pltpu.VMEM((C, D), jnp.float32),
        pltpu.VMEM((C, D), jnp.float32),
        pltpu.VMEM((D,), jnp.float32),
        pltpu.SemaphoreType.DMA,
        pltpu.SemaphoreType.DMA,
        pltpu.SemaphoreType.DMA,
        pltpu.SemaphoreType.DMA,
        pltpu.SemaphoreType.DMA,
        pltpu.SemaphoreType.DMA,
    ],
)
def _emb_kernel(idx_hbm, ww_hbm, wp_hbm, sc_hbm, out_hbm,
                idx_v, tok0_v, tok1_v, tok2_v, pos_v, scale_v,
                g0_sem, g1_sem, g2_sem, o0_sem, o1_sem, o2_sem):
    wid = lax.axis_index("s") * NC + lax.axis_index("c")
    t0 = wid * TW
    toks = (tok0_v, tok1_v, tok2_v)
    gsems = (g0_sem, g1_sem, g2_sem)
    osems = (o0_sem, o1_sem, o2_sem)

    def flat0(c):
        return (c & (B - 1)) * T + t0 + (c >> 2) * C

    pltpu.sync_copy(sc_hbm, scale_v)
    idx_copies = [
        pltpu.async_copy(idx_hbm.at[pl.ds(flat0(c), C)], idx_v.at[c], g2_sem)
        for c in range(NCHUNK)
    ]
    pltpu.sync_copy(wp_hbm.at[pl.ds(t0, C)], pos_v)
    for cp in idx_copies:
        cp.wait()
    scale_regs = [scale_v[pl.ds(j * L, L)] for j in range(NJ)]

    def start_gather(c):
        return pltpu.async_copy(ww_hbm.at[idx_v.at[c]],
                                toks[c % 3], gsems[c % 3])

    def compute(c):
        tok_v = toks[c % 3]

        def pair_body(p, cc):
            rows = [2 * p, 2 * p + 1]
            accs = [[jnp.zeros((L,), jnp.float32)] * 2 for _ in rows]
            for j in range(NJ):
                sl = pl.ds(j * L, L)
                for i, r in enumerate(rows):
                    x = tok_v[r, sl] + pos_v[r, sl]
                    tok_v[r, sl] = x
                    accs[i][j % 2] = accs[i][j % 2] + x * x
            rstds = []
            for i in range(len(rows)):
                acc = accs[i][0] + accs[i][1]
                acc = acc + lax.rev(acc, (0,))
                lanes = [acc[l] for l in range(L // 2)]
                while len(lanes) > 1:
                    lanes = [lanes[i2] + lanes[i2 + 1]
                             for i2 in range(0, len(lanes), 2)]
                ms = lanes[0] * (1.0 / D) + EPS
                rstds.append(jnp.full((L,), _rsqrt_scalar(ms), jnp.float32))
            for j in range(NJ):
                sl = pl.ds(j * L, L)
                for i, r in enumerate(rows):
                    tok_v[r, sl] = tok_v[r, sl] * rstds[i] * scale_regs[j]
            return cc

        lax.fori_loop(0, C // 2, pair_body, 0)

    gathers = {0: start_gather(0)}
    outs = {}
    for c in range(NCHUNK):
        if c + 1 < NCHUNK:
            if c - 2 in outs:
                outs[c - 2].wait()
            gathers[c + 1] = start_gather(c + 1)
        gathers[c].wait()
        if c == B:
            pltpu.sync_copy(wp_hbm.at[pl.ds(t0 + C, C)], pos_v)
        compute(c)
        outs[c] = pltpu.async_copy(toks[c % 3],
                                   out_hbm.at[pl.ds(flat0(c), C)],
                                   osems[c % 3])
    for c in range(max(0, NCHUNK - 3), NCHUNK):
        outs[c].wait()


def kernel(idx, W_word, W_pos, rms_scale):
    out = _emb_kernel(idx.reshape(N), W_word, W_pos, rms_scale)
    return out.reshape(B, T, D)


# speedup vs baseline: 2.3438x; 2.3438x over previous
import functools

import jax
import jax.numpy as jnp
from jax import lax
from jax.experimental import pallas as pl
from jax.experimental.pallas import tpu as pltpu
from jax.experimental.pallas import tpu_sc as plsc

D = 768
B = 4
T = 2048
N = B * T
EPS = 1e-6
NC, NS, L = 2, 16, 16
NW = NC * NS
PER_W = N // NW
C = 32
NCHUNK = PER_W // C
NJ = D // L
TW = T // NW


def _rsqrt_scalar(a):
    i = lax.bitcast_convert_type(a, jnp.int32)
    i = jnp.int32(0x5F3759DF) - (i >> 1)
    y = lax.bitcast_convert_type(i, jnp.float32)
    half_a = 0.5 * a
    for _ in range(2):
        y = y * (1.5 - half_a * y * y)
    return y


_mesh = plsc.VectorSubcoreMesh(core_axis_name="c", subcore_axis_name="s")


@functools.partial(
    pl.kernel,
    mesh=_mesh,
    out_type=jax.ShapeDtypeStruct((N, D), jnp.float32),
    scratch_types=[
        pltpu.VMEM((NCHUNK, C), jnp.int32),
        pltpu.VMEM((C, D), jnp.float32),
        pltpu.VMEM((C, D), jnp.float32),
        pltpu.VMEM((C, D), jnp.float32),
        pltpu.VMEM((C, D), jnp.float32),
        pltpu.VMEM((D,), jnp.float32),
        pltpu.SemaphoreType.DMA,
        pltpu.SemaphoreType.DMA,
        pltpu.SemaphoreType.DMA,
        pltpu.SemaphoreType.DMA,
        pltpu.SemaphoreType.DMA,
        pltpu.SemaphoreType.DMA,
    ],
)
def _emb_kernel(idx_hbm, ww_hbm, wp_hbm, sc_hbm, out_hbm,
                idx_v, tok0_v, tok1_v, tok2_v, pos_v, scale_v,
                g0_sem, g1_sem, g2_sem, o0_sem, o1_sem, o2_sem):
    wid = lax.axis_index("s") * NC + lax.axis_index("c")
    t0 = wid * TW
    toks = (tok0_v, tok1_v, tok2_v)
    gsems = (g0_sem, g1_sem, g2_sem)
    osems = (o0_sem, o1_sem, o2_sem)

    def flat0(c):
        return (c & (B - 1)) * T + t0 + (c >> 2) * C

    pltpu.sync_copy(sc_hbm, scale_v)
    idx_copies = [
        pltpu.async_copy(idx_hbm.at[pl.ds(flat0(c), C)], idx_v.at[c], g2_sem)
        for c in range(NCHUNK)
    ]
    pltpu.sync_copy(wp_hbm.at[pl.ds(t0, C)], pos_v)
    for cp in idx_copies:
        cp.wait()
    scale_regs = [scale_v[pl.ds(j * L, L)] for j in range(NJ)]

    def start_gather(c):
        return pltpu.async_copy(ww_hbm.at[idx_v.at[c]],
                                toks[c % 3], gsems[c % 3])

    def compute(c):
        tok_v = toks[c % 3]

        def pass1(r):
            accs = [jnp.zeros((L,), jnp.float32) for _ in range(4)]
            for j in range(NJ):
                sl = pl.ds(j * L, L)
                x = tok_v[r, sl] + pos_v[r, sl]
                tok_v[r, sl] = x
                accs[j % 4] = accs[j % 4] + x * x
            acc = (accs[0] + accs[1]) + (accs[2] + accs[3])
            acc = acc + lax.rev(acc, (0,))
            lanes = [acc[l] for l in range(L // 2)]
            while len(lanes) > 1:
                lanes = [lanes[i] + lanes[i + 1]
                         for i in range(0, len(lanes), 2)]
            ms = lanes[0] * (1.0 / D) + EPS
            return jnp.full((L,), _rsqrt_scalar(ms), jnp.float32)

        def pass2(r, rstd):
            for j in range(NJ):
                sl = pl.ds(j * L, L)
                tok_v[r, sl] = tok_v[r, sl] * rstd * scale_regs[j]

        def row_body(r, rstd_prev):
            pass2(r - 1, rstd_prev)
            return pass1(r)

        rstd0 = pass1(0)
        rstd_last = lax.fori_loop(1, C, row_body, rstd0)
        pass2(C - 1, rstd_last)

    gathers = {0: start_gather(0)}
    outs = {}
    for c in range(NCHUNK):
        if c + 1 < NCHUNK:
            if c - 2 in outs:
                outs[c - 2].wait()
            gathers[c + 1] = start_gather(c + 1)
        gathers[c].wait()
        if c == B:
            pltpu.sync_copy(wp_hbm.at[pl.ds(t0 + C, C)], pos_v)
        compute(c)
        outs[c] = pltpu.async_copy(toks[c % 3],
                                   out_hbm.at[pl.ds(flat0(c), C)],
                                   osems[c % 3])
    for c in range(max(0, NCHUNK - 3), NCHUNK):
        outs[c].wait()


def kernel(idx, W_word, W_pos, rms_scale):
    out = _emb_kernel(idx.reshape(N), W_word, W_pos, rms_scale)
    return out.reshape(B, T, D)


# speedup vs baseline: 2.4685x; 1.0532x over previous
import functools

import jax
import jax.numpy as jnp
from jax import lax
from jax.experimental import pallas as pl
from jax.experimental.pallas import tpu as pltpu
from jax.experimental.pallas import tpu_sc as plsc

D = 768
B = 4
T = 2048
N = B * T
EPS = 1e-6
NC, NS, L = 2, 16, 16
NW = NC * NS
PER_W = N // NW
C = 32
NCHUNK = PER_W // C
NJ = D // L
TW = T // NW


def _rsqrt_scalar(a):
    i = lax.bitcast_convert_type(a, jnp.int32)
    i = jnp.int32(0x5F3759DF) - (i >> 1)
    y = lax.bitcast_convert_type(i, jnp.float32)
    half_a = 0.5 * a
    for _ in range(2):
        y = y * (1.5 - half_a * y * y)
    return y


_mesh = plsc.VectorSubcoreMesh(core_axis_name="c", subcore_axis_name="s")


@functools.partial(
    pl.kernel,
    mesh=_mesh,
    out_type=jax.ShapeDtypeStruct((N, D), jnp.float32),
    scratch_types=[
        pltpu.VMEM((NCHUNK, C), jnp.int32),
        pltpu.VMEM((C, D), jnp.float32),
        pltpu.VMEM((C, D), jnp.float32),
        pltpu.VMEM((C, D), jnp.float32),
        pltpu.VMEM((C, D), jnp.float32),
        pltpu.VMEM((D,), jnp.float32),
        pltpu.SemaphoreType.DMA,
        pltpu.SemaphoreType.DMA,
        pltpu.SemaphoreType.DMA,
        pltpu.SemaphoreType.DMA,
        pltpu.SemaphoreType.DMA,
        pltpu.SemaphoreType.DMA,
    ],
)
def _emb_kernel(idx_hbm, ww_hbm, wp_hbm, sc_hbm, out_hbm,
                idx_v, tok0_v, tok1_v, tok2_v, pos_v, scale_v,
                g0_sem, g1_sem, g2_sem, o0_sem, o1_sem, o2_sem):
    wid = lax.axis_index("s") * NC + lax.axis_index("c")
    t0 = wid * TW
    toks = (tok0_v, tok1_v, tok2_v)
    gsems = (g0_sem, g1_sem, g2_sem)
    osems = (o0_sem, o1_sem, o2_sem)

    def flat0(c):
        return (c & (B - 1)) * T + t0 + (c >> 2) * C

    pltpu.sync_copy(sc_hbm, scale_v)
    idx_copies = [
        pltpu.async_copy(idx_hbm.at[pl.ds(flat0(c), C)], idx_v.at[c], g2_sem)
        for c in range(NCHUNK)
    ]
    pltpu.sync_copy(wp_hbm.at[pl.ds(t0, C)], pos_v)
    for cp in idx_copies:
        cp.wait()
    scale_regs = [scale_v[pl.ds(j * L, L)] for j in range(NJ)]

    def start_gather(c):
        return pltpu.async_copy(ww_hbm.at[idx_v.at[c]],
                                toks[c % 3], gsems[c % 3])

    def compute(c):
        tok_v = toks[c % 3]

        def pass1(r):
            accs = [jnp.zeros((L,), jnp.float32) for _ in range(4)]
            for j in range(NJ):
                sl = pl.ds(j * L, L)
                x = tok_v[r, sl] + pos_v[r, sl]
                tok_v[r, sl] = x
                accs[j % 4] = accs[j % 4] + x * x
            return (accs[0] + accs[1]) + (accs[2] + accs[3])

        def finalize(acc):
            acc = acc + lax.rev(acc, (0,))
            lanes = [acc[l] for l in range(L // 2)]
            while len(lanes) > 1:
                lanes = [lanes[i] + lanes[i + 1]
                         for i in range(0, len(lanes), 2)]
            ms = lanes[0] * (1.0 / D) + EPS
            return jnp.full((L,), _rsqrt_scalar(ms), jnp.float32)

        def pass2(r, rstd):
            for j in range(NJ):
                sl = pl.ds(j * L, L)
                tok_v[r, sl] = tok_v[r, sl] * rstd * scale_regs[j]

        def row_body(r, acc_prev):
            pass2(r - 1, finalize(acc_prev))
            return pass1(r)

        acc_last = lax.fori_loop(1, C, row_body, pass1(0))
        pass2(C - 1, finalize(acc_last))

    gathers = {0: start_gather(0)}
    outs = {}
    for c in range(NCHUNK):
        if c + 1 < NCHUNK:
            if c - 2 in outs:
                outs[c - 2].wait()
            gathers[c + 1] = start_gather(c + 1)
        gathers[c].wait()
        if c == B:
            pltpu.sync_copy(wp_hbm.at[pl.ds(t0 + C, C)], pos_v)
        compute(c)
        outs[c] = pltpu.async_copy(toks[c % 3],
                                   out_hbm.at[pl.ds(flat0(c), C)],
                                   osems[c % 3])
    for c in range(max(0, NCHUNK - 3), NCHUNK):
        outs[c].wait()


def kernel(idx, W_word, W_pos, rms_scale):
    out = _emb_kernel(idx.reshape(N), W_word, W_pos, rms_scale)
    return out.reshape(B, T, D)


# speedup vs baseline: 2.4755x; 1.0028x over previous
import functools

import jax
import jax.numpy as jnp
from jax import lax
from jax.experimental import pallas as pl
from jax.experimental.pallas import tpu as pltpu
from jax.experimental.pallas import tpu_sc as plsc

D = 768
B = 4
T = 2048
N = B * T
EPS = 1e-6
NC, NS, L = 2, 16, 16
NW = NC * NS
PER_W = N // NW
C = 32
NCHUNK = PER_W // C
NJ = D // L
TW = T // NW


def _rsqrt_scalar(a):
    i = lax.bitcast_convert_type(a, jnp.int32)
    i = jnp.int32(0x5F3759DF) - (i >> 1)
    y = lax.bitcast_convert_type(i, jnp.float32)
    half_a = 0.5 * a
    for _ in range(2):
        y = y * (1.5 - half_a * y * y)
    return y


_mesh = plsc.VectorSubcoreMesh(core_axis_name="c", subcore_axis_name="s")


@functools.partial(
    pl.kernel,
    mesh=_mesh,
    out_type=jax.ShapeDtypeStruct((N, D), jnp.float32),
    scratch_types=[
        pltpu.VMEM((NCHUNK, C), jnp.int32),
        pltpu.VMEM((C, D), jnp.float32),
        pltpu.VMEM((C, D), jnp.float32),
        pltpu.VMEM((C, D), jnp.float32),
        pltpu.VMEM((C, D), jnp.float32),
        pltpu.VMEM((D,), jnp.float32),
        pltpu.SemaphoreType.DMA,
        pltpu.SemaphoreType.DMA,
        pltpu.SemaphoreType.DMA,
        pltpu.SemaphoreType.DMA,
        pltpu.SemaphoreType.DMA,
        pltpu.SemaphoreType.DMA,
    ],
)
def _emb_kernel(idx_hbm, ww_hbm, wp_hbm, sc_hbm, out_hbm,
                idx_v, tok0_v, tok1_v, tok2_v, pos_v, scale_v,
                g0_sem, g1_sem, g2_sem, o0_sem, o1_sem, o2_sem):
    wid = lax.axis_index("s") * NC + lax.axis_index("c")
    t0 = wid * TW
    toks = (tok0_v, tok1_v, tok2_v)
    gsems = (g0_sem, g1_sem, g2_sem)
    osems = (o0_sem, o1_sem, o2_sem)

    def flat0(c):
        return (c & (B - 1)) * T + t0 + (c >> 2) * C

    cp0 = pltpu.async_copy(idx_hbm.at[pl.ds(flat0(0), C)], idx_v.at[0],
                           g0_sem)
    idx_copies = [
        pltpu.async_copy(idx_hbm.at[pl.ds(flat0(c), C)], idx_v.at[c], g2_sem)
        for c in range(1, NCHUNK)
    ]

    def start_gather(c):
        return pltpu.async_copy(ww_hbm.at[idx_v.at[c]],
                                toks[c % 3], gsems[c % 3])

    cp0.wait()
    gathers = {0: start_gather(0)}
    pltpu.sync_copy(wp_hbm.at[pl.ds(t0, C)], pos_v)
    pltpu.sync_copy(sc_hbm, scale_v)
    for cp in idx_copies:
        cp.wait()
    scale_regs = [scale_v[pl.ds(j * L, L)] for j in range(NJ)]

    def pass1(tv, r):
        accs = [jnp.zeros((L,), jnp.float32) for _ in range(4)]
        for j in range(NJ):
            sl = pl.ds(j * L, L)
            x = tv[r, sl] + pos_v[r, sl]
            tv[r, sl] = x
            accs[j % 4] = accs[j % 4] + x * x
        return (accs[0] + accs[1]) + (accs[2] + accs[3])

    def finalize(acc):
        acc = acc + lax.rev(acc, (0,))
        lanes = [acc[l] for l in range(L // 2)]
        while len(lanes) > 1:
            lanes = [lanes[i] + lanes[i + 1]
                     for i in range(0, len(lanes), 2)]
        ms = lanes[0] * (1.0 / D) + EPS
        return jnp.full((L,), _rsqrt_scalar(ms), jnp.float32)

    def pass2(tv, r, rstd):
        for j in range(NJ):
            sl = pl.ds(j * L, L)
            tv[r, sl] = tv[r, sl] * rstd * scale_regs[j]

    def out_copy(c):
        return pltpu.async_copy(toks[c % 3],
                                out_hbm.at[pl.ds(flat0(c), C)],
                                osems[c % 3])

    outs = {}
    carry = None
    for c in range(NCHUNK):
        if c + 1 < NCHUNK:
            if c - 2 in outs:
                outs[c - 2].wait()
            gathers[c + 1] = start_gather(c + 1)
        gathers[c].wait()
        if c == B:
            pltpu.sync_copy(wp_hbm.at[pl.ds(t0 + C, C)], pos_v)

        tok_v = toks[c % 3]
        if carry is not None:
            pass2(toks[(c - 1) % 3], C - 1, finalize(carry))
        acc0 = pass1(tok_v, 0)
        if c > 0:
            outs[c - 1] = out_copy(c - 1)

        def row_body(r, acc_prev, tv=tok_v):
            pass2(tv, r - 1, finalize(acc_prev))
            return pass1(tv, r)

        carry = lax.fori_loop(1, C, row_body, acc0)

    pass2(toks[(NCHUNK - 1) % 3], C - 1, finalize(carry))
    outs[NCHUNK - 1] = out_copy(NCHUNK - 1)
    for c in range(max(0, NCHUNK - 3), NCHUNK):
        outs[c].wait()


def kernel(idx, W_word, W_pos, rms_scale):
    out = _emb_kernel(idx.reshape(N), W_word, W_pos, rms_scale)
    return out.reshape(B, T, D)
